# baseline (speedup 1.0000x reference)
import jax
import jax.numpy as jnp
from jax.experimental import pallas as pl
from jax.experimental.pallas import tpu as pltpu

_TM = 512


def _prologue_kernel(H_ref, W_ref, b_ref, gamma_ref, beta_ref, lin_ref):
    h = H_ref[...]
    nrm = jnp.sqrt(jnp.sum(h * h, axis=1, keepdims=True))
    hn = h / jnp.maximum(nrm, 1e-12)
    mu = jnp.mean(hn, axis=0, keepdims=True)
    xc = hn - mu
    var = jnp.mean(xc * xc, axis=0, keepdims=True)
    hbn = xc * jax.lax.rsqrt(var + 1e-5) * gamma_ref[...] + beta_ref[...]
    lin = jax.lax.dot_general(
        hbn, W_ref[...], (((1,), (1,)), ((), ())),
        preferred_element_type=jnp.float32) + b_ref[...]
    lin_ref[...] = lin


def _spmm_kernel(lin_ref, A_ref, out_ref):
    acc = jnp.dot(A_ref[...], lin_ref[...],
                  preferred_element_type=jnp.float32,
                  precision=jax.lax.Precision.DEFAULT)
    out_ref[...] = jnp.where(acc >= 0, acc, 0.01 * acc)


def kernel(H, A_norm, W, b, gamma, beta):
    n, d_in = H.shape
    d_out = W.shape[0]
    lin = pl.pallas_call(
        _prologue_kernel,
        in_specs=[
            pl.BlockSpec((n, d_in), lambda: (0, 0)),
            pl.BlockSpec((d_out, d_in), lambda: (0, 0)),
            pl.BlockSpec((1, d_out), lambda: (0, 0)),
            pl.BlockSpec((1, d_in), lambda: (0, 0)),
            pl.BlockSpec((1, d_in), lambda: (0, 0)),
        ],
        out_specs=pl.BlockSpec((n, d_out), lambda: (0, 0)),
        out_shape=jax.ShapeDtypeStruct((n, d_out), jnp.float32),
    )(H, W, b.reshape(1, d_out), gamma.reshape(1, d_in),
      beta.reshape(1, d_in))
    return pl.pallas_call(
        _spmm_kernel,
        grid=(n // _TM,),
        in_specs=[
            pl.BlockSpec((n, d_out), lambda i: (0, 0)),
            pl.BlockSpec((_TM, n), lambda i: (i, 0)),
        ],
        out_specs=pl.BlockSpec((_TM, d_out), lambda i: (i, 0)),
        out_shape=jax.ShapeDtypeStruct((n, d_out), jnp.float32),
        compiler_params=pltpu.CompilerParams(
            dimension_semantics=("parallel",)),
    )(lin, A_norm)


# speedup vs baseline: 1.0693x; 1.0693x over previous
import jax
import jax.numpy as jnp
from jax.experimental import pallas as pl
from jax.experimental.pallas import tpu as pltpu

_TM = 512


def _fused_kernel(H_ref, W_ref, b_ref, gamma_ref, beta_ref, A1_ref, A2_ref,
                  out_ref, lin_ref):
    @pl.when(pl.program_id(0) == 0)
    def _prologue():
        h = H_ref[...]
        nrm = jnp.sqrt(jnp.sum(h * h, axis=1, keepdims=True))
        hn = h / jnp.maximum(nrm, 1e-12)
        mu = jnp.mean(hn, axis=0, keepdims=True)
        xc = hn - mu
        var = jnp.mean(xc * xc, axis=0, keepdims=True)
        hbn = xc * jax.lax.rsqrt(var + 1e-5) * gamma_ref[...] + beta_ref[...]
        lin = jax.lax.dot_general(
            hbn, W_ref[...], (((1,), (1,)), ((), ())),
            preferred_element_type=jnp.float32) + b_ref[...]
        lin_ref[...] = lin

    nh = lin_ref.shape[0] // 2
    acc = jnp.dot(A1_ref[...], lin_ref[:nh, :],
                  preferred_element_type=jnp.float32,
                  precision=jax.lax.Precision.DEFAULT)
    acc += jnp.dot(A2_ref[...], lin_ref[nh:, :],
                   preferred_element_type=jnp.float32,
                   precision=jax.lax.Precision.DEFAULT)
    out_ref[...] = jnp.where(acc >= 0, acc, 0.01 * acc)


def kernel(H, A_norm, W, b, gamma, beta):
    n, d_in = H.shape
    d_out = W.shape[0]
    nh = n // 2
    return pl.pallas_call(
        _fused_kernel,
        grid=(n // _TM,),
        in_specs=[
            pl.BlockSpec((n, d_in), lambda i: (0, 0)),
            pl.BlockSpec((d_out, d_in), lambda i: (0, 0)),
            pl.BlockSpec((1, d_out), lambda i: (0, 0)),
            pl.BlockSpec((1, d_in), lambda i: (0, 0)),
            pl.BlockSpec((1, d_in), lambda i: (0, 0)),
            pl.BlockSpec((_TM, nh), lambda i: (i, 0)),
            pl.BlockSpec((_TM, nh), lambda i: (i, 1)),
        ],
        out_specs=pl.BlockSpec((_TM, d_out), lambda i: (i, 0)),
        out_shape=jax.ShapeDtypeStruct((n, d_out), jnp.float32),
        scratch_shapes=[pltpu.VMEM((n, d_out), jnp.float32)],
        compiler_params=pltpu.CompilerParams(
            dimension_semantics=("arbitrary",)),
    )(H, W, b.reshape(1, d_out), gamma.reshape(1, d_in),
      beta.reshape(1, d_in), A_norm, A_norm)


# speedup vs baseline: 1.1270x; 1.0540x over previous
import jax
import jax.numpy as jnp
from jax.experimental import pallas as pl
from jax.experimental.pallas import tpu as pltpu

_TM = 1024


def _fused_kernel(H_ref, W_ref, b_ref, gamma_ref, beta_ref, A_ref,
                  out_ref, lin_ref):
    @pl.when(pl.program_id(0) == 0)
    def _prologue():
        h = H_ref[...]
        nrm = jnp.sqrt(jnp.sum(h * h, axis=1, keepdims=True))
        hn = h / jnp.maximum(nrm, 1e-12)
        mu = jnp.mean(hn, axis=0, keepdims=True)
        xc = hn - mu
        var = jnp.mean(xc * xc, axis=0, keepdims=True)
        hbn = xc * jax.lax.rsqrt(var + 1e-5) * gamma_ref[...] + beta_ref[...]
        lin = jax.lax.dot_general(
            hbn, W_ref[...], (((1,), (1,)), ((), ())),
            preferred_element_type=jnp.float32) + b_ref[...]
        lin_ref[...] = lin

    acc = jnp.dot(A_ref[...], lin_ref[...],
                  preferred_element_type=jnp.float32,
                  precision=jax.lax.Precision.DEFAULT)
    out_ref[...] = jnp.where(acc >= 0, acc, 0.01 * acc)


def kernel(H, A_norm, W, b, gamma, beta):
    n, d_in = H.shape
    d_out = W.shape[0]
    return pl.pallas_call(
        _fused_kernel,
        grid=(n // _TM,),
        in_specs=[
            pl.BlockSpec((n, d_in), lambda i: (0, 0)),
            pl.BlockSpec((d_out, d_in), lambda i: (0, 0)),
            pl.BlockSpec((1, d_out), lambda i: (0, 0)),
            pl.BlockSpec((1, d_in), lambda i: (0, 0)),
            pl.BlockSpec((1, d_in), lambda i: (0, 0)),
            pl.BlockSpec((_TM, n), lambda i: (i, 0)),
        ],
        out_specs=pl.BlockSpec((_TM, d_out), lambda i: (i, 0)),
        out_shape=jax.ShapeDtypeStruct((n, d_out), jnp.float32),
        scratch_shapes=[pltpu.VMEM((n, d_out), jnp.float32)],
        compiler_params=pltpu.CompilerParams(
            dimension_semantics=("arbitrary",)),
    )(H, W, b.reshape(1, d_out), gamma.reshape(1, d_in),
      beta.reshape(1, d_in), A_norm)
